# baseline (speedup 1.0000x reference)
import jax
import jax.numpy as jnp
from jax import lax
from jax.experimental import pallas as pl
from jax.experimental.pallas import tpu as pltpu
from jax.experimental.pallas import tpu_sc as plsc

_D = 64
_S = 200
_B = 4096
_NW = 32
_CHUNK = _B // _NW
_LANES = 16
_NBUF = 4


def _positional_table():
    position = jnp.arange(0, _S, dtype=jnp.float32).reshape((_S, 1))
    even_i = jnp.arange(0, _D, 2, dtype=jnp.float32)
    odd_i = jnp.arange(1, _D, 2, dtype=jnp.float32)
    pow_even = jnp.power(10000.0, 2.0 * even_i / _D)
    pow_odd = jnp.power(10000.0, 2.0 * odd_i / _D)
    pe_even = jnp.sin(position / pow_even)
    pe_odd = jnp.cos(position / pow_odd)
    pe = jnp.stack([pe_even, pe_odd], axis=2).reshape(_S, _D)
    return pe


def _body(idx_hbm, pe_hbm, tab_hbm, out_hbm,
          pe_v, iraw, rows, obuf, si, sg, sw):
    wid = lax.axis_index("s") * 2 + lax.axis_index("c")
    bcol = wid * _CHUNK
    pltpu.sync_copy(pe_hbm, pe_v)

    def fire_idx(u, b):
        pltpu.async_copy(idx_hbm.at[u, pl.ds(bcol, _CHUNK)], iraw.at[b], si[b])

    def drain_idx(b):
        pltpu.make_async_copy(idx_hbm.at[0, pl.ds(0, _CHUNK)], iraw.at[b], si[b]).wait()

    def fire_gather(b):
        pltpu.async_copy(tab_hbm.at[iraw.at[b]], rows.at[b], sg[b])

    def drain_gather(b):
        pltpu.make_async_copy(tab_hbm.at[pl.ds(0, _CHUNK)], rows.at[b], sg[b]).wait()

    def drain_writeout(b):
        pltpu.make_async_copy(obuf.at[b], out_hbm.at[0, :, pl.ds(0, _CHUNK)], sw[b]).wait()

    def compute(u, b):
        ulo = jnp.broadcast_to(u, (_LANES,)).astype(jnp.int32)

        @pl.loop(0, _D)
        def _d(d):
            dlo = jnp.broadcast_to(d, (_LANES,)).astype(jnp.int32)
            pes = plsc.load_gather(pe_v, [ulo, dlo])
            for j in range(_CHUNK // _LANES):
                sl = pl.ds(j * _LANES, _LANES)
                riota = lax.iota(jnp.int32, _LANES) + j * _LANES
                v = plsc.load_gather(rows.at[b], [riota, dlo])
                obuf[b, d, sl] = v + pes

    fire_idx(0, 0)
    fire_idx(1, 1)
    drain_idx(0)
    fire_gather(0)

    @pl.loop(0, _S, step=_NBUF)
    def _block(t):
        for k in range(_NBUF):
            u = t + k
            b2 = (k + 2) % _NBUF
            b1 = (k + 1) % _NBUF

            @pl.when(u < _S - 2)
            def _stage_idx():
                fire_idx(u + 2, b2)

            @pl.when(u < _S - 1)
            def _stage_gather():
                drain_idx(b1)
                fire_gather(b1)

            drain_gather(k)

            @pl.when(u >= _NBUF)
            def _drain_w():
                drain_writeout(k)

            compute(u, k)
            pltpu.async_copy(obuf.at[k], out_hbm.at[u, :, pl.ds(bcol, _CHUNK)], sw[k])

    for k in range(_NBUF):
        drain_writeout(k)


@jax.jit
def kernel(data, table):
    idx_t = data.T.astype(jnp.int32)
    pe = _positional_table()
    mesh = plsc.VectorSubcoreMesh(core_axis_name="c", subcore_axis_name="s")
    out_k = pl.kernel(
        _body,
        out_type=jax.ShapeDtypeStruct((_S, _D, _B), jnp.float32),
        mesh=mesh,
        scratch_types=[
            pltpu.VMEM((_S, _D), jnp.float32),
            pltpu.VMEM((_NBUF, _CHUNK), jnp.int32),
            pltpu.VMEM((_NBUF, _CHUNK, _D), jnp.float32),
            pltpu.VMEM((_NBUF, _D, _CHUNK), jnp.float32),
            [pltpu.SemaphoreType.DMA] * _NBUF,
            [pltpu.SemaphoreType.DMA] * _NBUF,
            [pltpu.SemaphoreType.DMA] * _NBUF,
        ],
        compiler_params=pltpu.CompilerParams(
            needs_layout_passes=False, use_tc_tiling_on_sc=False),
    )(idx_t, pe, table)
    return out_k.transpose(2, 0, 1)


# speedup vs baseline: 1.5664x; 1.5664x over previous
import jax
import jax.numpy as jnp
from jax import lax
from jax.experimental import pallas as pl
from jax.experimental.pallas import tpu as pltpu
from jax.experimental.pallas import tpu_sc as plsc

_D = 64
_S = 200
_B = 4096
_NW = 32
_CHUNK = _B // _NW
_LANES = 16
_NBUF = 4
_CHP = _CHUNK + 1


def _positional_table():
    position = jnp.arange(0, _S, dtype=jnp.float32).reshape((_S, 1))
    even_i = jnp.arange(0, _D, 2, dtype=jnp.float32)
    odd_i = jnp.arange(1, _D, 2, dtype=jnp.float32)
    pow_even = jnp.power(10000.0, 2.0 * even_i / _D)
    pow_odd = jnp.power(10000.0, 2.0 * odd_i / _D)
    pe_even = jnp.sin(position / pow_even)
    pe_odd = jnp.cos(position / pow_odd)
    pe = jnp.stack([pe_even, pe_odd], axis=2).reshape(_S, _D)
    return pe


def _body(idx_hbm, pe_hbm, tab_hbm, out_hbm,
          pe_v, iraw, rows, obuf, si, sg, sw):
    wid = lax.axis_index("s") * 2 + lax.axis_index("c")
    bcol = wid * _CHUNK
    pltpu.sync_copy(pe_hbm, pe_v)

    def fire_idx(u, b):
        pltpu.async_copy(idx_hbm.at[u, pl.ds(bcol, _CHUNK)], iraw.at[b], si[b])

    def drain_idx(b):
        pltpu.make_async_copy(idx_hbm.at[0, pl.ds(0, _CHUNK)], iraw.at[b], si[b]).wait()

    def fire_gather(b):
        pltpu.async_copy(tab_hbm.at[iraw.at[b]], rows.at[b], sg[b])

    def drain_gather(b):
        pltpu.make_async_copy(tab_hbm.at[pl.ds(0, _CHUNK)], rows.at[b], sg[b]).wait()

    def drain_writeout(b):
        pltpu.make_async_copy(obuf.at[b, :, pl.ds(0, _CHUNK)],
                              out_hbm.at[0, :, pl.ds(0, _CHUNK)], sw[b]).wait()

    def compute(u, b):
        pes = [pe_v[u, pl.ds(c * _LANES, _LANES)] for c in range(_D // _LANES)]

        @pl.loop(0, _CHUNK)
        def _i(i):
            ibc = jnp.broadcast_to(i, (_LANES,)).astype(jnp.int32)
            for c in range(_D // _LANES):
                diota = lax.iota(jnp.int32, _LANES) + c * _LANES
                v = rows[b, i, pl.ds(c * _LANES, _LANES)]
                plsc.store_scatter(obuf.at[b], [diota, ibc], v + pes[c])

    fire_idx(0, 0)
    fire_idx(1, 1)
    drain_idx(0)
    fire_gather(0)

    @pl.loop(0, _S, step=_NBUF)
    def _block(t):
        for k in range(_NBUF):
            u = t + k
            b2 = (k + 2) % _NBUF
            b1 = (k + 1) % _NBUF

            @pl.when(u < _S - 2)
            def _stage_idx():
                fire_idx(u + 2, b2)

            @pl.when(u < _S - 1)
            def _stage_gather():
                drain_idx(b1)
                fire_gather(b1)

            drain_gather(k)

            @pl.when(u >= _NBUF)
            def _drain_w():
                drain_writeout(k)

            compute(u, k)
            pltpu.async_copy(obuf.at[k, :, pl.ds(0, _CHUNK)],
                             out_hbm.at[u, :, pl.ds(bcol, _CHUNK)], sw[k])

    for k in range(_NBUF):
        drain_writeout(k)


@jax.jit
def kernel(data, table):
    idx_t = data.T.astype(jnp.int32)
    pe = _positional_table()
    mesh = plsc.VectorSubcoreMesh(core_axis_name="c", subcore_axis_name="s")
    out_k = pl.kernel(
        _body,
        out_type=jax.ShapeDtypeStruct((_S, _D, _B), jnp.float32),
        mesh=mesh,
        scratch_types=[
            pltpu.VMEM((_S, _D), jnp.float32),
            pltpu.VMEM((_NBUF, _CHUNK), jnp.int32),
            pltpu.VMEM((_NBUF, _CHUNK, _D), jnp.float32),
            pltpu.VMEM((_NBUF, _D, _CHP), jnp.float32),
            [pltpu.SemaphoreType.DMA] * _NBUF,
            [pltpu.SemaphoreType.DMA] * _NBUF,
            [pltpu.SemaphoreType.DMA] * _NBUF,
        ],
        compiler_params=pltpu.CompilerParams(
            needs_layout_passes=False, use_tc_tiling_on_sc=False),
    )(idx_t, pe, table)
    return out_k.transpose(2, 0, 1)


# speedup vs baseline: 1.6956x; 1.0825x over previous
import jax
import jax.numpy as jnp
from jax import lax
from jax.experimental import pallas as pl
from jax.experimental.pallas import tpu as pltpu
from jax.experimental.pallas import tpu_sc as plsc

_D = 64
_S = 200
_B = 4096
_NW = 32
_CHUNK = _B // _NW
_LANES = 16
_NBUF = 4
_CHP = _CHUNK + 1


def _positional_table():
    position = jnp.arange(0, _S, dtype=jnp.float32).reshape((_S, 1))
    even_i = jnp.arange(0, _D, 2, dtype=jnp.float32)
    odd_i = jnp.arange(1, _D, 2, dtype=jnp.float32)
    pow_even = jnp.power(10000.0, 2.0 * even_i / _D)
    pow_odd = jnp.power(10000.0, 2.0 * odd_i / _D)
    pe_even = jnp.sin(position / pow_even)
    pe_odd = jnp.cos(position / pow_odd)
    pe = jnp.stack([pe_even, pe_odd], axis=2).reshape(_S, _D)
    return pe


def _body(idx_hbm, pe_hbm, tab_hbm, out_hbm,
          pe_v, iraw, rows, si, sg, sw):
    wid = lax.axis_index("s") * 2 + lax.axis_index("c")
    bcol = wid * _CHUNK
    pltpu.sync_copy(pe_hbm, pe_v)

    def fire_idx(u, b):
        pltpu.async_copy(idx_hbm.at[u, pl.ds(bcol, _CHUNK)], iraw.at[b], si[b])

    def drain_idx(b):
        pltpu.make_async_copy(idx_hbm.at[0, pl.ds(0, _CHUNK)], iraw.at[b], si[b]).wait()

    def fire_gather(b):
        pltpu.async_copy(tab_hbm.at[iraw.at[b]], rows.at[b], sg[b])

    def drain_gather(b):
        pltpu.make_async_copy(tab_hbm.at[pl.ds(0, _CHUNK)], rows.at[b], sg[b]).wait()

    def drain_writeout(b):
        pltpu.make_async_copy(rows.at[b], out_hbm.at[pl.ds(0, _CHUNK), 0, :],
                              sw[b]).wait()

    def compute(u, b):
        pes = [pe_v[u, pl.ds(c * _LANES, _LANES)] for c in range(_D // _LANES)]

        @pl.loop(0, _CHUNK)
        def _i(i):
            for c in range(_D // _LANES):
                sl = pl.ds(c * _LANES, _LANES)
                rows[b, i, sl] = rows[b, i, sl] + pes[c]

    fire_idx(0, 0)
    fire_idx(1, 1)
    drain_idx(0)
    fire_gather(0)

    @pl.loop(0, _S, step=_NBUF)
    def _block(t):
        for k in range(_NBUF):
            u = t + k
            b2 = (k + 2) % _NBUF
            b1 = (k + 1) % _NBUF

            @pl.when(u < _S - 2)
            def _stage_idx():
                fire_idx(u + 2, b2)

            @pl.when(u < _S - 1)
            def _stage_gather():
                @pl.when(u >= _NBUF - 1)
                def _drain_prev():
                    drain_writeout(b1)
                drain_idx(b1)
                fire_gather(b1)

            drain_gather(k)
            compute(u, k)
            pltpu.async_copy(rows.at[k], out_hbm.at[pl.ds(bcol, _CHUNK), u, :],
                             sw[k])

    for k in range(_NBUF):
        drain_writeout(k)


@jax.jit
def kernel(data, table):
    idx_t = data.T.astype(jnp.int32)
    pe = _positional_table()
    mesh = plsc.VectorSubcoreMesh(core_axis_name="c", subcore_axis_name="s")
    out_k = pl.kernel(
        _body,
        out_type=jax.ShapeDtypeStruct((_B, _S, _D), jnp.float32),
        mesh=mesh,
        scratch_types=[
            pltpu.VMEM((_S, _D), jnp.float32),
            pltpu.VMEM((_NBUF, _CHUNK), jnp.int32),
            pltpu.VMEM((_NBUF, _CHUNK, _D), jnp.float32),
            [pltpu.SemaphoreType.DMA] * _NBUF,
            [pltpu.SemaphoreType.DMA] * _NBUF,
            [pltpu.SemaphoreType.DMA] * _NBUF,
        ],
        compiler_params=pltpu.CompilerParams(
            needs_layout_passes=False, use_tc_tiling_on_sc=False),
    )(idx_t, pe, table)
    return out_k
